# Initial kernel scaffold; baseline (speedup 1.0000x reference)
import functools

import jax
import jax.numpy as jnp
from jax import lax
from jax.experimental import pallas as pl
from jax.experimental.pallas import tpu as pltpu
from jax.experimental.pallas import tpu_sc as plsc

_NC = 2
_NS = 16
_B = 80


def _tables_body(h_ref, wms_ref, wmr_ref, ts_ref, tr_ref):
    h = h_ref[...]
    ts_ref[...] = jnp.dot(h, wms_ref[...], preferred_element_type=jnp.float32)
    tr_ref[...] = jnp.dot(h, wmr_ref[...], preferred_element_type=jnp.float32)


def _ew_body(e_ref, wme_ref, b_ref, out_ref):
    out_ref[...] = (
        jnp.dot(e_ref[...], wme_ref[...], preferred_element_type=jnp.float32)
        + b_ref[...]
    )


def _hnew_body(h_ref, a0_ref, a1_ref, wnh_ref, wna_ref, b_ref, out_ref):
    acc = jnp.dot(h_ref[...], wnh_ref[...], preferred_element_type=jnp.float32)
    acc += jnp.dot(a0_ref[0] + a1_ref[0], wna_ref[...],
                   preferred_element_type=jnp.float32)
    out_ref[...] = jnp.maximum(acc + b_ref[...], 0.0)


def _enew_body(e_ref, m_ref, wee_ref, wem_ref, b_ref, out_ref):
    acc = jnp.dot(e_ref[...], wee_ref[...], preferred_element_type=jnp.float32)
    acc += jnp.dot(m_ref[...], wem_ref[...], preferred_element_type=jnp.float32)
    out_ref[...] = jnp.maximum(acc + b_ref[...], 0.0)


@functools.lru_cache(maxsize=None)
def _make_sc_messages(E, N, F):
    per_tile = E // (_NC * _NS)
    assert per_tile * _NC * _NS == E
    n_chunks = per_tile // _B
    assert n_chunks * _B == per_tile
    rows_per_sub = N // _NS
    assert rows_per_sub * _NS == N

    mesh = plsc.VectorSubcoreMesh(core_axis_name="c", subcore_axis_name="s")

    @functools.partial(
        pl.kernel,
        out_type=[
            jax.ShapeDtypeStruct((E, F), jnp.float32),
            jax.ShapeDtypeStruct((_NC, N, F), jnp.float32),
        ],
        mesh=mesh,
        scratch_types=[
            pltpu.VMEM((_B,), jnp.int32),
            pltpu.VMEM((_B,), jnp.int32),
            pltpu.VMEM((_B, F), jnp.float32),
            pltpu.VMEM((_B, F), jnp.float32),
            pltpu.VMEM((_B, F), jnp.float32),
            pltpu.VMEM_SHARED((N, F), jnp.float32),
            pltpu.SemaphoreType.DMA,
            pltpu.SemaphoreType.DMA,
        ],
    )
    def sc_messages(ew, snd, rcv, ts, tr, zeros, m_out, agg_out,
                    idx_s, idx_r, buf_ew, buf_s, buf_r, agg_sh, sem_a, sem_b):
        cid = lax.axis_index("c")
        sid = lax.axis_index("s")
        wid = cid * _NS + sid

        r0 = sid * rows_per_sub
        pltpu.sync_copy(zeros.at[pl.ds(r0, rows_per_sub)],
                        agg_sh.at[pl.ds(r0, rows_per_sub)])
        plsc.subcore_barrier()

        base0 = wid * per_tile

        def chunk_body(c, carry):
            base = base0 + c * _B
            pltpu.sync_copy(snd.at[pl.ds(base, _B)], idx_s)
            pltpu.sync_copy(rcv.at[pl.ds(base, _B)], idx_r)
            cp_ew = pltpu.async_copy(ew.at[pl.ds(base, _B)], buf_ew, sem_a)
            cp_s = pltpu.async_copy(ts.at[idx_s], buf_s, sem_b)
            cp_r = pltpu.async_copy(tr.at[idx_r], buf_r, sem_b)
            cp_ew.wait()
            cp_s.wait()
            cp_r.wait()

            def row_body(i, rc):
                for j in range(F // 16):
                    sl = pl.ds(j * 16, 16)
                    v = buf_ew[i, sl] + buf_s[i, sl] + buf_r[i, sl]
                    buf_ew[i, sl] = jnp.maximum(v, 0.0)
                return rc

            lax.fori_loop(0, _B, row_body, 0)

            pltpu.sync_copy(buf_ew, m_out.at[pl.ds(base, _B)])
            pltpu.sync_copy(buf_ew, agg_sh.at[idx_r], add=True)
            return carry

        lax.fori_loop(0, n_chunks, chunk_body, 0)

        plsc.subcore_barrier()
        pltpu.sync_copy(agg_sh.at[pl.ds(r0, rows_per_sub)],
                        agg_out.at[cid, pl.ds(r0, rows_per_sub)])

    return sc_messages


def kernel(h, e, senders, receivers, W_msg, b_msg, W_node, b_node, W_edge, b_edge):
    N, F = h.shape
    E, De = e.shape
    senders = senders.astype(jnp.int32)
    receivers = receivers.astype(jnp.int32)

    wm_e = W_msg[:De]
    wm_s = W_msg[De:De + F]
    wm_r = W_msg[De + F:]
    wn_h = W_node[:F]
    wn_a = W_node[F:]
    we_e = W_edge[:De]
    we_m = W_edge[De:]
    b_msg2 = b_msg.reshape(1, F)
    b_node2 = b_node.reshape(1, F)
    b_edge2 = b_edge.reshape(1, De)

    bn = 1000
    be = 4000

    ts, tr = pl.pallas_call(
        _tables_body,
        grid=(N // bn,),
        in_specs=[
            pl.BlockSpec((bn, F), lambda i: (i, 0)),
            pl.BlockSpec((F, F), lambda i: (0, 0)),
            pl.BlockSpec((F, F), lambda i: (0, 0)),
        ],
        out_specs=[
            pl.BlockSpec((bn, F), lambda i: (i, 0)),
            pl.BlockSpec((bn, F), lambda i: (i, 0)),
        ],
        out_shape=[
            jax.ShapeDtypeStruct((N, F), jnp.float32),
            jax.ShapeDtypeStruct((N, F), jnp.float32),
        ],
    )(h, wm_s, wm_r)

    ew = pl.pallas_call(
        _ew_body,
        grid=(E // be,),
        in_specs=[
            pl.BlockSpec((be, De), lambda i: (i, 0)),
            pl.BlockSpec((De, F), lambda i: (0, 0)),
            pl.BlockSpec((1, F), lambda i: (0, 0)),
        ],
        out_specs=pl.BlockSpec((be, F), lambda i: (i, 0)),
        out_shape=jax.ShapeDtypeStruct((E, F), jnp.float32),
    )(e, wm_e, b_msg2)

    zeros = jnp.zeros((N, F), jnp.float32)
    m, agg_parts = _make_sc_messages(E, N, F)(
        ew, senders, receivers, ts, tr, zeros)

    h_new = pl.pallas_call(
        _hnew_body,
        grid=(N // bn,),
        in_specs=[
            pl.BlockSpec((bn, F), lambda i: (i, 0)),
            pl.BlockSpec((1, bn, F), lambda i: (0, i, 0)),
            pl.BlockSpec((1, bn, F), lambda i: (1, i, 0)),
            pl.BlockSpec((F, F), lambda i: (0, 0)),
            pl.BlockSpec((F, F), lambda i: (0, 0)),
            pl.BlockSpec((1, F), lambda i: (0, 0)),
        ],
        out_specs=pl.BlockSpec((bn, F), lambda i: (i, 0)),
        out_shape=jax.ShapeDtypeStruct((N, F), jnp.float32),
    )(h, agg_parts, agg_parts, wn_h, wn_a, b_node2)

    e_new = pl.pallas_call(
        _enew_body,
        grid=(E // be,),
        in_specs=[
            pl.BlockSpec((be, De), lambda i: (i, 0)),
            pl.BlockSpec((be, F), lambda i: (i, 0)),
            pl.BlockSpec((De, De), lambda i: (0, 0)),
            pl.BlockSpec((F, De), lambda i: (0, 0)),
            pl.BlockSpec((1, De), lambda i: (0, 0)),
        ],
        out_specs=pl.BlockSpec((be, De), lambda i: (i, 0)),
        out_shape=jax.ShapeDtypeStruct((E, De), jnp.float32),
    )(e, m, we_e, we_m, b_edge2)

    return h_new, e_new


# speedup vs baseline: 2.5527x; 2.5527x over previous
import functools

import jax
import jax.numpy as jnp
from jax import lax
from jax.experimental import pallas as pl
from jax.experimental.pallas import tpu as pltpu
from jax.experimental.pallas import tpu_sc as plsc

_NC = 2
_NS = 16
_B = 80


def _tables_body(h_ref, wms_ref, wmr_ref, ts_ref, tr_ref):
    h = h_ref[...]
    ts_ref[...] = jnp.dot(h, wms_ref[...], preferred_element_type=jnp.float32)
    tr_ref[...] = jnp.dot(h, wmr_ref[...], preferred_element_type=jnp.float32)


def _ew_body(e_ref, wme_ref, b_ref, out_ref):
    out_ref[...] = (
        jnp.dot(e_ref[...], wme_ref[...], preferred_element_type=jnp.float32)
        + b_ref[...]
    )


def _hnew_body(h_ref, a0_ref, a1_ref, wnh_ref, wna_ref, b_ref, out_ref):
    acc = jnp.dot(h_ref[...], wnh_ref[...], preferred_element_type=jnp.float32)
    acc += jnp.dot(a0_ref[0] + a1_ref[0], wna_ref[...],
                   preferred_element_type=jnp.float32)
    out_ref[...] = jnp.maximum(acc + b_ref[...], 0.0)


def _enew_body(e_ref, m_ref, wee_ref, wem_ref, b_ref, out_ref):
    acc = jnp.dot(e_ref[...], wee_ref[...], preferred_element_type=jnp.float32)
    acc += jnp.dot(m_ref[...], wem_ref[...], preferred_element_type=jnp.float32)
    out_ref[...] = jnp.maximum(acc + b_ref[...], 0.0)


@functools.lru_cache(maxsize=None)
def _make_sc_messages(E, N_pad, F):
    per_tile = E // (_NC * _NS)
    assert per_tile * _NC * _NS == E
    n_chunks = per_tile // _B
    assert n_chunks * _B == per_tile
    rows_per_sub = N_pad // _NS
    assert rows_per_sub * _NS == N_pad and rows_per_sub % 8 == 0

    mesh = plsc.VectorSubcoreMesh(core_axis_name="c", subcore_axis_name="s",
                                  num_cores=_NC, num_subcores=_NS)

    @functools.partial(
        pl.kernel,
        out_type=[
            jax.ShapeDtypeStruct((E, F), jnp.float32),
            jax.ShapeDtypeStruct((_NC, N_pad, F), jnp.float32),
        ],
        mesh=mesh,
        scratch_types=[
            pltpu.VMEM((_B,), jnp.int32),
            pltpu.VMEM((_B,), jnp.int32),
            pltpu.VMEM((_B, F), jnp.float32),
            pltpu.VMEM((_B, F), jnp.float32),
            pltpu.VMEM((_B, F), jnp.float32),
            pltpu.VMEM_SHARED((N_pad, F), jnp.float32),
            pltpu.SemaphoreType.DMA,
            pltpu.SemaphoreType.DMA,
        ],
    )
    def sc_messages(ew, snd, rcv, ts, tr, zeros, m_out, agg_out,
                    idx_s, idx_r, buf_ew, buf_s, buf_r, agg_sh, sem_a, sem_b):
        cid = lax.axis_index("c")
        sid = lax.axis_index("s")
        wid = cid * _NS + sid

        r0 = sid * rows_per_sub
        pltpu.sync_copy(zeros.at[pl.ds(r0, rows_per_sub)],
                        agg_sh.at[pl.ds(r0, rows_per_sub)])
        plsc.subcore_barrier()

        base0 = wid * per_tile

        def chunk_body(c, carry):
            base = base0 + c * _B
            pltpu.sync_copy(snd.at[pl.ds(base, _B)], idx_s)
            pltpu.sync_copy(rcv.at[pl.ds(base, _B)], idx_r)
            cp_ew = pltpu.async_copy(ew.at[pl.ds(base, _B)], buf_ew, sem_a)
            cp_s = pltpu.async_copy(ts.at[idx_s], buf_s, sem_b)
            cp_r = pltpu.async_copy(tr.at[idx_r], buf_r, sem_b)
            cp_ew.wait()
            cp_s.wait()
            cp_r.wait()

            def row_body(i, rc):
                for j in range(F // 16):
                    sl = pl.ds(j * 16, 16)
                    v = buf_ew[i, sl] + buf_s[i, sl] + buf_r[i, sl]
                    buf_ew[i, sl] = jnp.maximum(v, 0.0)
                return rc

            lax.fori_loop(0, _B, row_body, 0)

            pltpu.sync_copy(buf_ew, m_out.at[pl.ds(base, _B)])
            pltpu.sync_copy(buf_ew, agg_sh.at[idx_r], add=True)
            return carry

        lax.fori_loop(0, n_chunks, chunk_body, 0)

        plsc.subcore_barrier()
        pltpu.sync_copy(agg_sh.at[pl.ds(r0, rows_per_sub)],
                        agg_out.at[cid, pl.ds(r0, rows_per_sub)])

    return sc_messages


def kernel(h, e, senders, receivers, W_msg, b_msg, W_node, b_node, W_edge, b_edge):
    N, F = h.shape
    E, De = e.shape
    senders = senders.astype(jnp.int32)
    receivers = receivers.astype(jnp.int32)

    wm_e = W_msg[:De]
    wm_s = W_msg[De:De + F]
    wm_r = W_msg[De + F:]
    wn_h = W_node[:F]
    wn_a = W_node[F:]
    we_e = W_edge[:De]
    we_m = W_edge[De:]
    b_msg2 = b_msg.reshape(1, F)
    b_node2 = b_node.reshape(1, F)
    b_edge2 = b_edge.reshape(1, De)

    bn = 1000
    be = 4000

    ts, tr = pl.pallas_call(
        _tables_body,
        grid=(N // bn,),
        in_specs=[
            pl.BlockSpec((bn, F), lambda i: (i, 0)),
            pl.BlockSpec((F, F), lambda i: (0, 0)),
            pl.BlockSpec((F, F), lambda i: (0, 0)),
        ],
        out_specs=[
            pl.BlockSpec((bn, F), lambda i: (i, 0)),
            pl.BlockSpec((bn, F), lambda i: (i, 0)),
        ],
        out_shape=[
            jax.ShapeDtypeStruct((N, F), jnp.float32),
            jax.ShapeDtypeStruct((N, F), jnp.float32),
        ],
    )(h, wm_s, wm_r)

    ew = pl.pallas_call(
        _ew_body,
        grid=(E // be,),
        in_specs=[
            pl.BlockSpec((be, De), lambda i: (i, 0)),
            pl.BlockSpec((De, F), lambda i: (0, 0)),
            pl.BlockSpec((1, F), lambda i: (0, 0)),
        ],
        out_specs=pl.BlockSpec((be, F), lambda i: (i, 0)),
        out_shape=jax.ShapeDtypeStruct((E, F), jnp.float32),
    )(e, wm_e, b_msg2)

    n_pad = ((N + (8 * _NS) - 1) // (8 * _NS)) * (8 * _NS)
    zeros = jnp.zeros((n_pad, F), jnp.float32)
    m, agg_parts = _make_sc_messages(E, n_pad, F)(
        ew, senders, receivers, ts, tr, zeros)

    h_new = pl.pallas_call(
        _hnew_body,
        grid=(N // bn,),
        in_specs=[
            pl.BlockSpec((bn, F), lambda i: (i, 0)),
            pl.BlockSpec((1, bn, F), lambda i: (0, i, 0)),
            pl.BlockSpec((1, bn, F), lambda i: (1, i, 0)),
            pl.BlockSpec((F, F), lambda i: (0, 0)),
            pl.BlockSpec((F, F), lambda i: (0, 0)),
            pl.BlockSpec((1, F), lambda i: (0, 0)),
        ],
        out_specs=pl.BlockSpec((bn, F), lambda i: (i, 0)),
        out_shape=jax.ShapeDtypeStruct((N, F), jnp.float32),
    )(h, agg_parts, agg_parts, wn_h, wn_a, b_node2)

    e_new = pl.pallas_call(
        _enew_body,
        grid=(E // be,),
        in_specs=[
            pl.BlockSpec((be, De), lambda i: (i, 0)),
            pl.BlockSpec((be, F), lambda i: (i, 0)),
            pl.BlockSpec((De, De), lambda i: (0, 0)),
            pl.BlockSpec((F, De), lambda i: (0, 0)),
            pl.BlockSpec((1, De), lambda i: (0, 0)),
        ],
        out_specs=pl.BlockSpec((be, De), lambda i: (i, 0)),
        out_shape=jax.ShapeDtypeStruct((E, De), jnp.float32),
    )(e, m, we_e, we_m, b_edge2)

    return h_new, e_new


# speedup vs baseline: 3.2574x; 1.2761x over previous
import functools

import jax
import jax.numpy as jnp
from jax import lax
from jax.experimental import pallas as pl
from jax.experimental.pallas import tpu as pltpu
from jax.experimental.pallas import tpu_sc as plsc

_NC = 2
_NS = 16
_B = 40


def _tables_body(h_ref, wms_ref, wmr_ref, ts_ref, tr_ref):
    h = h_ref[...]
    ts_ref[...] = jnp.dot(h, wms_ref[...], preferred_element_type=jnp.float32)
    tr_ref[...] = jnp.dot(h, wmr_ref[...], preferred_element_type=jnp.float32)


def _ew_body(e_ref, wme_ref, b_ref, out_ref):
    out_ref[...] = (
        jnp.dot(e_ref[...], wme_ref[...], preferred_element_type=jnp.float32)
        + b_ref[...]
    )


def _hnew_body(h_ref, a0_ref, a1_ref, wnh_ref, wna_ref, b_ref, out_ref):
    acc = jnp.dot(h_ref[...], wnh_ref[...], preferred_element_type=jnp.float32)
    acc += jnp.dot(a0_ref[0] + a1_ref[0], wna_ref[...],
                   preferred_element_type=jnp.float32)
    out_ref[...] = jnp.maximum(acc + b_ref[...], 0.0)


def _enew_body(e_ref, m_ref, wee_ref, wem_ref, b_ref, out_ref):
    acc = jnp.dot(e_ref[...], wee_ref[...], preferred_element_type=jnp.float32)
    acc += jnp.dot(m_ref[...], wem_ref[...], preferred_element_type=jnp.float32)
    out_ref[...] = jnp.maximum(acc + b_ref[...], 0.0)


@functools.lru_cache(maxsize=None)
def _make_sc_messages(E, N_pad, F):
    per_tile = E // (_NC * _NS)
    assert per_tile * _NC * _NS == E
    n_chunks = per_tile // _B
    assert n_chunks * _B == per_tile and n_chunks % 2 == 0
    rows_per_sub = N_pad // _NS
    assert rows_per_sub * _NS == N_pad and rows_per_sub % 8 == 0

    mesh = plsc.VectorSubcoreMesh(core_axis_name="c", subcore_axis_name="s",
                                  num_cores=_NC, num_subcores=_NS)

    @functools.partial(
        pl.kernel,
        out_type=[
            jax.ShapeDtypeStruct((E, F), jnp.float32),
            jax.ShapeDtypeStruct((_NC, N_pad, F), jnp.float32),
        ],
        mesh=mesh,
        scratch_types=[
            pltpu.VMEM((_B,), jnp.int32),
            pltpu.VMEM((_B,), jnp.int32),
            pltpu.VMEM((_B,), jnp.int32),
            pltpu.VMEM((_B,), jnp.int32),
            pltpu.VMEM((_B, F), jnp.float32),
            pltpu.VMEM((_B, F), jnp.float32),
            pltpu.VMEM((_B, F), jnp.float32),
            pltpu.VMEM((_B, F), jnp.float32),
            pltpu.VMEM((_B, F), jnp.float32),
            pltpu.VMEM((_B, F), jnp.float32),
            pltpu.VMEM_SHARED((N_pad, F), jnp.float32),
            pltpu.SemaphoreType.DMA,
            pltpu.SemaphoreType.DMA,
            pltpu.SemaphoreType.DMA,
            pltpu.SemaphoreType.DMA,
            pltpu.SemaphoreType.DMA,
            pltpu.SemaphoreType.DMA,
        ],
    )
    def sc_messages(ew, snd, rcv, ts, tr, zeros, m_out, agg_out,
                    idx_s0, idx_s1, idx_r0, idx_r1,
                    buf_ew0, buf_ew1, buf_s0, buf_s1, buf_r0, buf_r1,
                    agg_sh, isem0, isem1, gsem0, gsem1, msem0, msem1):
        idx_s = (idx_s0, idx_s1)
        idx_r = (idx_r0, idx_r1)
        buf_ew = (buf_ew0, buf_ew1)
        buf_s = (buf_s0, buf_s1)
        buf_r = (buf_r0, buf_r1)
        isem = (isem0, isem1)
        gsem = (gsem0, gsem1)
        msem = (msem0, msem1)
        cid = lax.axis_index("c")
        sid = lax.axis_index("s")
        wid = cid * _NS + sid
        base0 = wid * per_tile

        r0 = sid * rows_per_sub
        pltpu.sync_copy(zeros.at[pl.ds(r0, rows_per_sub)],
                        agg_sh.at[pl.ds(r0, rows_per_sub)])
        plsc.subcore_barrier()

        def issue_idx(c, slot):
            base = base0 + c * _B
            pltpu.async_copy(snd.at[pl.ds(base, _B)], idx_s[slot], isem[slot])
            pltpu.async_copy(rcv.at[pl.ds(base, _B)], idx_r[slot], isem[slot])

        def wait_idx(slot):
            pltpu.make_async_copy(snd.at[pl.ds(0, _B)], idx_s[slot],
                                  isem[slot]).wait()
            pltpu.make_async_copy(rcv.at[pl.ds(0, _B)], idx_r[slot],
                                  isem[slot]).wait()

        def issue_data(c, slot):
            pltpu.async_copy(ew.at[pl.ds(base0 + c * _B, _B)], buf_ew[slot],
                             gsem[slot])
            pltpu.async_copy(ts.at[idx_s[slot]], buf_s[slot], gsem[slot])
            pltpu.async_copy(tr.at[idx_r[slot]], buf_r[slot], gsem[slot])

        def wait_data(slot):
            pltpu.make_async_copy(ew.at[pl.ds(0, _B)], buf_ew[slot],
                                  gsem[slot]).wait()
            pltpu.make_async_copy(ts.at[pl.ds(0, _B)], buf_s[slot],
                                  gsem[slot]).wait()
            pltpu.make_async_copy(tr.at[pl.ds(0, _B)], buf_r[slot],
                                  gsem[slot]).wait()

        def step(g, k, o):
            @pl.when(g >= 1)
            def _():
                pltpu.make_async_copy(buf_ew[o], m_out.at[pl.ds(0, _B)],
                                      msem[o]).wait()

            @pl.when(g + 1 < n_chunks)
            def _():
                wait_idx(o)
                issue_data(g + 1, o)

            wait_data(k)

            def row_body(i, rc):
                for j in range(F // 16):
                    sl = pl.ds(j * 16, 16)
                    v = buf_ew[k][i, sl] + buf_s[k][i, sl] + buf_r[k][i, sl]
                    buf_ew[k][i, sl] = jnp.maximum(v, 0.0)
                return rc

            lax.fori_loop(0, _B, row_body, 0)
            pltpu.async_copy(buf_ew[k], m_out.at[pl.ds(base0 + g * _B, _B)],
                             msem[k])
            pltpu.sync_copy(buf_ew[k], agg_sh.at[idx_r[k]], add=True)

            @pl.when(g + 2 < n_chunks)
            def _():
                issue_idx(g + 2, k)

        issue_idx(0, 0)
        issue_idx(1, 1)
        wait_idx(0)
        issue_data(0, 0)

        def pair_body(t, carry):
            g0 = 2 * t
            step(g0, 0, 1)
            step(g0 + 1, 1, 0)
            return carry

        lax.fori_loop(0, n_chunks // 2, pair_body, 0)
        last_slot = (n_chunks - 1) % 2
        pltpu.make_async_copy(buf_ew[last_slot], m_out.at[pl.ds(0, _B)],
                              msem[last_slot]).wait()

        plsc.subcore_barrier()
        pltpu.sync_copy(agg_sh.at[pl.ds(r0, rows_per_sub)],
                        agg_out.at[cid, pl.ds(r0, rows_per_sub)])

    return sc_messages


def kernel(h, e, senders, receivers, W_msg, b_msg, W_node, b_node, W_edge, b_edge):
    N, F = h.shape
    E, De = e.shape
    senders = senders.astype(jnp.int32)
    receivers = receivers.astype(jnp.int32)

    wm_e = W_msg[:De]
    wm_s = W_msg[De:De + F]
    wm_r = W_msg[De + F:]
    wn_h = W_node[:F]
    wn_a = W_node[F:]
    we_e = W_edge[:De]
    we_m = W_edge[De:]
    b_msg2 = b_msg.reshape(1, F)
    b_node2 = b_node.reshape(1, F)
    b_edge2 = b_edge.reshape(1, De)

    bn = 1000
    be = 4000

    ts, tr = pl.pallas_call(
        _tables_body,
        grid=(N // bn,),
        in_specs=[
            pl.BlockSpec((bn, F), lambda i: (i, 0)),
            pl.BlockSpec((F, F), lambda i: (0, 0)),
            pl.BlockSpec((F, F), lambda i: (0, 0)),
        ],
        out_specs=[
            pl.BlockSpec((bn, F), lambda i: (i, 0)),
            pl.BlockSpec((bn, F), lambda i: (i, 0)),
        ],
        out_shape=[
            jax.ShapeDtypeStruct((N, F), jnp.float32),
            jax.ShapeDtypeStruct((N, F), jnp.float32),
        ],
    )(h, wm_s, wm_r)

    ew = pl.pallas_call(
        _ew_body,
        grid=(E // be,),
        in_specs=[
            pl.BlockSpec((be, De), lambda i: (i, 0)),
            pl.BlockSpec((De, F), lambda i: (0, 0)),
            pl.BlockSpec((1, F), lambda i: (0, 0)),
        ],
        out_specs=pl.BlockSpec((be, F), lambda i: (i, 0)),
        out_shape=jax.ShapeDtypeStruct((E, F), jnp.float32),
    )(e, wm_e, b_msg2)

    n_pad = ((N + (8 * _NS) - 1) // (8 * _NS)) * (8 * _NS)
    zeros = jnp.zeros((n_pad, F), jnp.float32)
    m, agg_parts = _make_sc_messages(E, n_pad, F)(
        ew, senders, receivers, ts, tr, zeros)

    h_new = pl.pallas_call(
        _hnew_body,
        grid=(N // bn,),
        in_specs=[
            pl.BlockSpec((bn, F), lambda i: (i, 0)),
            pl.BlockSpec((1, bn, F), lambda i: (0, i, 0)),
            pl.BlockSpec((1, bn, F), lambda i: (1, i, 0)),
            pl.BlockSpec((F, F), lambda i: (0, 0)),
            pl.BlockSpec((F, F), lambda i: (0, 0)),
            pl.BlockSpec((1, F), lambda i: (0, 0)),
        ],
        out_specs=pl.BlockSpec((bn, F), lambda i: (i, 0)),
        out_shape=jax.ShapeDtypeStruct((N, F), jnp.float32),
    )(h, agg_parts, agg_parts, wn_h, wn_a, b_node2)

    e_new = pl.pallas_call(
        _enew_body,
        grid=(E // be,),
        in_specs=[
            pl.BlockSpec((be, De), lambda i: (i, 0)),
            pl.BlockSpec((be, F), lambda i: (i, 0)),
            pl.BlockSpec((De, De), lambda i: (0, 0)),
            pl.BlockSpec((F, De), lambda i: (0, 0)),
            pl.BlockSpec((1, De), lambda i: (0, 0)),
        ],
        out_specs=pl.BlockSpec((be, De), lambda i: (i, 0)),
        out_shape=jax.ShapeDtypeStruct((E, De), jnp.float32),
    )(e, m, we_e, we_m, b_edge2)

    return h_new, e_new
